# Initial kernel scaffold; baseline (speedup 1.0000x reference)
import jax
import jax.numpy as jnp
from jax.experimental import pallas as pl


def kernel(x, row_index, src_index, att_bias, dist, pos, src_pos, org_to_src, Wq, bq, Wk, bk, g_att, b_att, g_mlp, b_mlp, W_in, b_in, W_out, b_out):
    raise NotImplementedError("write your pallas kernel here")


# speedup vs baseline: 27.1327x; 27.1327x over previous
import functools

import jax
import jax.numpy as jnp
from jax import lax
from jax.experimental import pallas as pl
from jax.experimental.pallas import tpu as pltpu
from jax.experimental.pallas import tpu_sc as plsc

N = 10000
E = 320000
S = 10000
D = 128
H = 8
DH = 16
HID = 256
NPAD = 10240
CW = 128

NC = 2
NS = 16
NW = NC * NS
CHUNK = 80

_mesh = plsc.VectorSubcoreMesh(core_axis_name="c", subcore_axis_name="s")


def _sc_gather(table, idx, d):
    b = idx.shape[0]
    per_w = b // NW
    n_chunks = per_w // CHUNK

    @functools.partial(
        pl.kernel,
        mesh=_mesh,
        out_type=jax.ShapeDtypeStruct((b, d), jnp.float32),
        scratch_types=[
            pltpu.VMEM((CHUNK,), jnp.int32),
            pltpu.VMEM((CHUNK, d), jnp.float32),
            pltpu.SemaphoreType.DMA,
        ],
    )
    def gk(table_hbm, idx_hbm, out_hbm, idx_v, rows_v, sem):
        wid = lax.axis_index("s") * NC + lax.axis_index("c")
        base = wid * per_w

        def body(j, carry):
            off = base + j * CHUNK
            pltpu.sync_copy(idx_hbm.at[pl.ds(off, CHUNK)], idx_v)
            pltpu.async_copy(table_hbm.at[idx_v], rows_v, sem).wait()
            pltpu.sync_copy(rows_v, out_hbm.at[pl.ds(off, CHUNK)])
            return carry

        lax.fori_loop(0, n_chunks, body, 0)

    return gk(table, idx)


def _sc_scatter_add(vals, idx, zeros):
    per_w = E // NW
    n_chunks = per_w // CHUNK
    rows_per_s = NPAD // NS

    @functools.partial(
        pl.kernel,
        mesh=_mesh,
        out_type=jax.ShapeDtypeStruct((2 * NPAD, CW), jnp.float32),
        scratch_types=[
            pltpu.VMEM((CHUNK,), jnp.int32),
            pltpu.VMEM((CHUNK, CW), jnp.float32),
            pltpu.VMEM_SHARED((NPAD, CW), jnp.float32),
        ],
    )
    def sk(vals_hbm, idx_hbm, zeros_hbm, out_hbm, idx_v, rows_v, acc_sh):
        cid = lax.axis_index("c")
        sid = lax.axis_index("s")
        wid = sid * NC + cid
        base = wid * per_w
        pltpu.sync_copy(
            zeros_hbm.at[pl.ds(sid * rows_per_s, rows_per_s)],
            acc_sh.at[pl.ds(sid * rows_per_s, rows_per_s)],
        )
        plsc.subcore_barrier()

        def body(j, carry):
            off = base + j * CHUNK
            pltpu.sync_copy(idx_hbm.at[pl.ds(off, CHUNK)], idx_v)
            pltpu.sync_copy(vals_hbm.at[pl.ds(off, CHUNK)], rows_v)
            pltpu.sync_copy(rows_v, acc_sh.at[idx_v], add=True)
            return carry

        lax.fori_loop(0, n_chunks, body, 0)
        plsc.subcore_barrier()
        pltpu.sync_copy(
            acc_sh.at[pl.ds(sid * rows_per_s, rows_per_s)],
            out_hbm.at[pl.ds(cid * NPAD + sid * rows_per_s, rows_per_s)],
        )

    return sk(vals, idx, zeros)


def _k1_body(x_ref, wq_ref, wk_ref, bq_ref, bk_ref, ga_ref, ba_ref, gm_ref,
             bm_ref, q_ref, kf_ref, zm_ref):
    x = x_ref[...]
    mu = jnp.mean(x, axis=-1, keepdims=True)
    var = jnp.mean((x - mu) * (x - mu), axis=-1, keepdims=True)
    xn = (x - mu) / jnp.sqrt(var + 1e-5)
    z = xn * ga_ref[...] + ba_ref[...]
    zm_ref[...] = xn * gm_ref[...] + bm_ref[...]
    dn = (((1,), (1,)), ((), ()))
    q = lax.dot_general(z, wq_ref[...], dn, preferred_element_type=jnp.float32)
    q = (q + bq_ref[...]) * 0.25
    kf = lax.dot_general(z, wk_ref[...], dn, preferred_element_type=jnp.float32)
    q_ref[...] = q
    kf_ref[...] = kf + bk_ref[...]


def _k2_body(qe_ref, kespe_ref, bias_ref, dist_ref, g_ref, out_ref):
    qe = qe_ref[...]
    ke = kespe_ref[:, :D]
    sp = kespe_ref[:, D:D + 3]
    p = qe * ke
    l8 = lax.dot_general(p, g_ref[...], (((1,), (0,)), ((), ())),
                         preferred_element_type=jnp.float32)
    ex = jnp.exp(l8 + bias_ref[...])
    d = dist_ref[...]
    invd = jnp.where(d == 0.0, 0.0, 1.0 / jnp.where(d == 0.0, 1.0, d))
    w = ex * invd
    pad = jnp.zeros((ex.shape[0], CW - 40), jnp.float32)
    out_ref[...] = jnp.concatenate(
        [ex, w, w * sp[:, 0:1], w * sp[:, 1:2], w * sp[:, 2:3], pad], axis=1)


def _k3_body(acc_a_ref, acc_b_ref, x_ref, zm_ref, pos_ref, w1_ref, w2_ref,
             bin_ref, wo_ref, bo_ref, out_ref):
    acc = acc_a_ref[0] + acc_b_ref[0]
    den = acc[:, 0:8]
    wn = acc[:, 8:16]
    nx = acc[:, 16:24]
    ny = acc[:, 24:32]
    nz = acc[:, 32:40]
    sd = jnp.where(den == 0.0, 1.0, den)
    rn = wn / sd
    fx = nx / sd - rn * pos_ref[:, 0:1]
    fy = ny / sd - rn * pos_ref[:, 1:2]
    fz = nz / sd - rn * pos_ref[:, 2:3]
    feat = jnp.concatenate([fx, fy, fz], axis=1)
    dn = (((1,), (1,)), ((), ()))
    hh = lax.dot_general(zm_ref[...], w1_ref[...], dn,
                         preferred_element_type=jnp.float32)
    hh = hh + lax.dot_general(feat, w2_ref[...], dn,
                              preferred_element_type=jnp.float32)
    hh = jnp.maximum(hh + bin_ref[...], 0.0)
    out = lax.dot_general(hh, wo_ref[...], dn,
                          preferred_element_type=jnp.float32)
    out_ref[...] = x_ref[...] + out + bo_ref[...]


def kernel(x, row_index, src_index, att_bias, dist, pos, src_pos, org_to_src,
           Wq, bq, Wk, bk, g_att, b_att, g_mlp, b_mlp, W_in, b_in, W_out,
           b_out):
    f32 = jnp.float32
    row_i = row_index.astype(jnp.int32)
    src_i = src_index.astype(jnp.int32)
    o2s = org_to_src.astype(jnp.int32)

    RB = 1000
    row1 = lambda v: v.reshape(1, -1)
    q, kf, zm = pl.pallas_call(
        _k1_body,
        grid=(N // RB,),
        in_specs=[
            pl.BlockSpec((RB, D), lambda i: (i, 0)),
            pl.BlockSpec((D, D), lambda i: (0, 0)),
            pl.BlockSpec((D, D), lambda i: (0, 0)),
            pl.BlockSpec((1, D), lambda i: (0, 0)),
            pl.BlockSpec((1, D), lambda i: (0, 0)),
            pl.BlockSpec((1, D), lambda i: (0, 0)),
            pl.BlockSpec((1, D), lambda i: (0, 0)),
            pl.BlockSpec((1, D), lambda i: (0, 0)),
            pl.BlockSpec((1, D), lambda i: (0, 0)),
        ],
        out_specs=[
            pl.BlockSpec((RB, D), lambda i: (i, 0)),
            pl.BlockSpec((RB, D), lambda i: (i, 0)),
            pl.BlockSpec((RB, D), lambda i: (i, 0)),
        ],
        out_shape=[
            jax.ShapeDtypeStruct((N, D), f32),
            jax.ShapeDtypeStruct((N, D), f32),
            jax.ShapeDtypeStruct((N, D), f32),
        ],
    )(x, Wq, Wk, row1(bq), row1(bk), row1(g_att), row1(b_att), row1(g_mlp),
      row1(b_mlp))

    o2s_pad = jnp.concatenate([o2s, jnp.zeros((NPAD - S,), jnp.int32)])
    ks = _sc_gather(kf, o2s_pad, D)
    kspe_tab = jnp.concatenate(
        [ks[:S], src_pos.astype(f32), jnp.zeros((S, 125), f32)], axis=1)
    kespe = _sc_gather(kspe_tab, src_i, 2 * D)
    qe = _sc_gather(q, row_i, D)

    EB = 2000
    gmat = (jnp.arange(D)[:, None] // DH == jnp.arange(H)[None, :]).astype(f32)
    bias_t = att_bias.T.astype(f32)
    dist_c = dist.astype(f32).reshape(E, 1)
    vals = pl.pallas_call(
        _k2_body,
        grid=(E // EB,),
        in_specs=[
            pl.BlockSpec((EB, D), lambda i: (i, 0)),
            pl.BlockSpec((EB, 2 * D), lambda i: (i, 0)),
            pl.BlockSpec((EB, H), lambda i: (i, 0)),
            pl.BlockSpec((EB, 1), lambda i: (i, 0)),
            pl.BlockSpec((D, H), lambda i: (0, 0)),
        ],
        out_specs=pl.BlockSpec((EB, CW), lambda i: (i, 0)),
        out_shape=jax.ShapeDtypeStruct((E, CW), f32),
    )(qe, kespe, bias_t, dist_c, gmat)

    acc2 = _sc_scatter_add(vals, row_i, jnp.zeros((NPAD, CW), f32))
    acc3 = acc2.reshape(2, NPAD, CW)

    pos16 = jnp.concatenate([pos.astype(f32), jnp.zeros((N, 13), f32)], axis=1)
    w1 = W_in[:, :D]
    perm = jnp.array([D + 3 * h + c for c in range(3) for h in range(H)],
                     jnp.int32)
    w2 = W_in[:, perm]
    out = pl.pallas_call(
        _k3_body,
        grid=(N // RB,),
        in_specs=[
            pl.BlockSpec((1, RB, CW), lambda i: (0, i, 0)),
            pl.BlockSpec((1, RB, CW), lambda i: (1, i, 0)),
            pl.BlockSpec((RB, D), lambda i: (i, 0)),
            pl.BlockSpec((RB, D), lambda i: (i, 0)),
            pl.BlockSpec((RB, 16), lambda i: (i, 0)),
            pl.BlockSpec((HID, D), lambda i: (0, 0)),
            pl.BlockSpec((HID, 24), lambda i: (0, 0)),
            pl.BlockSpec((1, HID), lambda i: (0, 0)),
            pl.BlockSpec((D, HID), lambda i: (0, 0)),
            pl.BlockSpec((1, D), lambda i: (0, 0)),
        ],
        out_specs=pl.BlockSpec((RB, D), lambda i: (i, 0)),
        out_shape=jax.ShapeDtypeStruct((N, D), f32),
    )(acc3, acc3, x, zm, pos16, w1, w2, row1(b_in), W_out, row1(b_out))
    return out


# speedup vs baseline: 30.7208x; 1.1322x over previous
import functools

import jax
import jax.numpy as jnp
from jax import lax
from jax.experimental import pallas as pl
from jax.experimental.pallas import tpu as pltpu
from jax.experimental.pallas import tpu_sc as plsc

N = 10000
E = 320000
S = 10000
D = 128
H = 8
DH = 16
HID = 256
NPAD = 10240
CW = 128

NC = 2
NS = 16
NW = NC * NS
CHUNK = 128

_mesh = plsc.VectorSubcoreMesh(core_axis_name="c", subcore_axis_name="s")


def _sc_gather(table, idx, d):
    b = idx.shape[0]
    per_w = b // NW
    n_full = per_w // CHUNK
    rem = per_w % CHUNK

    scratch = [
        pltpu.VMEM((CHUNK,), jnp.int32),
        pltpu.VMEM((CHUNK, d), jnp.float32),
        pltpu.SemaphoreType.DMA,
    ]
    if rem:
        scratch += [
            pltpu.VMEM((rem,), jnp.int32),
            pltpu.VMEM((rem, d), jnp.float32),
        ]

    @functools.partial(
        pl.kernel,
        mesh=_mesh,
        out_type=jax.ShapeDtypeStruct((b, d), jnp.float32),
        scratch_types=scratch,
    )
    def gk(table_hbm, idx_hbm, out_hbm, idx_v, rows_v, sem, *tail):
        wid = lax.axis_index("s") * NC + lax.axis_index("c")
        base = wid * per_w

        def body(j, carry):
            off = base + j * CHUNK
            pltpu.sync_copy(idx_hbm.at[pl.ds(off, CHUNK)], idx_v)
            pltpu.async_copy(table_hbm.at[idx_v], rows_v, sem).wait()
            pltpu.sync_copy(rows_v, out_hbm.at[pl.ds(off, CHUNK)])
            return carry

        lax.fori_loop(0, n_full, body, 0)
        if rem:
            idx_t, rows_t = tail
            off = base + n_full * CHUNK
            pltpu.sync_copy(idx_hbm.at[pl.ds(off, rem)], idx_t)
            pltpu.async_copy(table_hbm.at[idx_t], rows_t, sem).wait()
            pltpu.sync_copy(rows_t, out_hbm.at[pl.ds(off, rem)])

    return gk(table, idx)


def _sc_scatter_add(vals, idx, zeros):
    per_w = E // NW
    n_full = per_w // CHUNK
    rem = per_w % CHUNK
    rows_per_s = NPAD // NS

    scratch = [
        pltpu.VMEM((CHUNK,), jnp.int32),
        pltpu.VMEM((CHUNK, CW), jnp.float32),
        pltpu.VMEM_SHARED((NPAD, CW), jnp.float32),
    ]
    if rem:
        scratch += [
            pltpu.VMEM((rem,), jnp.int32),
            pltpu.VMEM((rem, CW), jnp.float32),
        ]

    @functools.partial(
        pl.kernel,
        mesh=_mesh,
        out_type=jax.ShapeDtypeStruct((2 * NPAD, CW), jnp.float32),
        scratch_types=scratch,
    )
    def sk(vals_hbm, idx_hbm, zeros_hbm, out_hbm, idx_v, rows_v, acc_sh,
           *tail):
        cid = lax.axis_index("c")
        sid = lax.axis_index("s")
        wid = sid * NC + cid
        base = wid * per_w
        pltpu.sync_copy(
            zeros_hbm.at[pl.ds(sid * rows_per_s, rows_per_s)],
            acc_sh.at[pl.ds(sid * rows_per_s, rows_per_s)],
        )
        plsc.subcore_barrier()

        def body(j, carry):
            off = base + j * CHUNK
            pltpu.sync_copy(idx_hbm.at[pl.ds(off, CHUNK)], idx_v)
            pltpu.sync_copy(vals_hbm.at[pl.ds(off, CHUNK)], rows_v)
            pltpu.sync_copy(rows_v, acc_sh.at[idx_v], add=True)
            return carry

        lax.fori_loop(0, n_full, body, 0)
        if rem:
            idx_t, rows_t = tail
            off = base + n_full * CHUNK
            pltpu.sync_copy(idx_hbm.at[pl.ds(off, rem)], idx_t)
            pltpu.sync_copy(vals_hbm.at[pl.ds(off, rem)], rows_t)
            pltpu.sync_copy(rows_t, acc_sh.at[idx_t], add=True)
        plsc.subcore_barrier()
        pltpu.sync_copy(
            acc_sh.at[pl.ds(sid * rows_per_s, rows_per_s)],
            out_hbm.at[pl.ds(cid * NPAD + sid * rows_per_s, rows_per_s)],
        )

    return sk(vals, idx, zeros)


def _k1_body(x_ref, wq_ref, wk_ref, bq_ref, bk_ref, ga_ref, ba_ref, gm_ref,
             bm_ref, q_ref, kf_ref, zm_ref):
    x = x_ref[...]
    mu = jnp.mean(x, axis=-1, keepdims=True)
    var = jnp.mean((x - mu) * (x - mu), axis=-1, keepdims=True)
    xn = (x - mu) / jnp.sqrt(var + 1e-5)
    z = xn * ga_ref[...] + ba_ref[...]
    zm_ref[...] = xn * gm_ref[...] + bm_ref[...]
    dn = (((1,), (1,)), ((), ()))
    q = lax.dot_general(z, wq_ref[...], dn, preferred_element_type=jnp.float32)
    q = (q + bq_ref[...]) * 0.25
    kf = lax.dot_general(z, wk_ref[...], dn, preferred_element_type=jnp.float32)
    q_ref[...] = q
    kf_ref[...] = kf + bk_ref[...]


def _k2_body(qe_ref, kespe_ref, bias_ref, dist_ref, g_ref, out_ref):
    qe = qe_ref[...]
    ke = kespe_ref[:, :D]
    sp = kespe_ref[:, D:D + 3]
    p = qe * ke
    l8 = lax.dot_general(p, g_ref[...], (((1,), (0,)), ((), ())),
                         preferred_element_type=jnp.float32)
    ex = jnp.exp(l8 + bias_ref[...])
    d = dist_ref[...]
    invd = jnp.where(d == 0.0, 0.0, 1.0 / jnp.where(d == 0.0, 1.0, d))
    w = ex * invd
    pad = jnp.zeros((ex.shape[0], CW - 40), jnp.float32)
    out_ref[...] = jnp.concatenate(
        [ex, w, w * sp[:, 0:1], w * sp[:, 1:2], w * sp[:, 2:3], pad], axis=1)


def _k3_body(acc_a_ref, acc_b_ref, x_ref, zm_ref, pos_ref, w1_ref, w2_ref,
             bin_ref, wo_ref, bo_ref, out_ref):
    acc = acc_a_ref[0] + acc_b_ref[0]
    den = acc[:, 0:8]
    wn = acc[:, 8:16]
    nx = acc[:, 16:24]
    ny = acc[:, 24:32]
    nz = acc[:, 32:40]
    sd = jnp.where(den == 0.0, 1.0, den)
    rn = wn / sd
    fx = nx / sd - rn * pos_ref[:, 0:1]
    fy = ny / sd - rn * pos_ref[:, 1:2]
    fz = nz / sd - rn * pos_ref[:, 2:3]
    feat = jnp.concatenate([fx, fy, fz], axis=1)
    dn = (((1,), (1,)), ((), ()))
    hh = lax.dot_general(zm_ref[...], w1_ref[...], dn,
                         preferred_element_type=jnp.float32)
    hh = hh + lax.dot_general(feat, w2_ref[...], dn,
                              preferred_element_type=jnp.float32)
    hh = jnp.maximum(hh + bin_ref[...], 0.0)
    out = lax.dot_general(hh, wo_ref[...], dn,
                          preferred_element_type=jnp.float32)
    out_ref[...] = x_ref[...] + out + bo_ref[...]


def kernel(x, row_index, src_index, att_bias, dist, pos, src_pos, org_to_src,
           Wq, bq, Wk, bk, g_att, b_att, g_mlp, b_mlp, W_in, b_in, W_out,
           b_out):
    f32 = jnp.float32
    row_i = row_index.astype(jnp.int32)
    src_i = src_index.astype(jnp.int32)
    o2s = org_to_src.astype(jnp.int32)

    RB = 1000
    row1 = lambda v: v.reshape(1, -1)
    q, kf, zm = pl.pallas_call(
        _k1_body,
        grid=(N // RB,),
        in_specs=[
            pl.BlockSpec((RB, D), lambda i: (i, 0)),
            pl.BlockSpec((D, D), lambda i: (0, 0)),
            pl.BlockSpec((D, D), lambda i: (0, 0)),
            pl.BlockSpec((1, D), lambda i: (0, 0)),
            pl.BlockSpec((1, D), lambda i: (0, 0)),
            pl.BlockSpec((1, D), lambda i: (0, 0)),
            pl.BlockSpec((1, D), lambda i: (0, 0)),
            pl.BlockSpec((1, D), lambda i: (0, 0)),
            pl.BlockSpec((1, D), lambda i: (0, 0)),
        ],
        out_specs=[
            pl.BlockSpec((RB, D), lambda i: (i, 0)),
            pl.BlockSpec((RB, D), lambda i: (i, 0)),
            pl.BlockSpec((RB, D), lambda i: (i, 0)),
        ],
        out_shape=[
            jax.ShapeDtypeStruct((N, D), f32),
            jax.ShapeDtypeStruct((N, D), f32),
            jax.ShapeDtypeStruct((N, D), f32),
        ],
    )(x, Wq, Wk, row1(bq), row1(bk), row1(g_att), row1(b_att), row1(g_mlp),
      row1(b_mlp))

    o2s_pad = jnp.concatenate([o2s, jnp.zeros((NPAD - S,), jnp.int32)])
    ks = _sc_gather(kf, o2s_pad, D)
    kspe_tab = jnp.concatenate(
        [ks[:S], src_pos.astype(f32), jnp.zeros((S, 125), f32)], axis=1)
    kespe = _sc_gather(kspe_tab, src_i, 2 * D)
    qe = _sc_gather(q, row_i, D)

    EB = 2000
    gmat = (jnp.arange(D)[:, None] // DH == jnp.arange(H)[None, :]).astype(f32)
    bias_t = att_bias.T.astype(f32)
    dist_c = dist.astype(f32).reshape(E, 1)
    vals = pl.pallas_call(
        _k2_body,
        grid=(E // EB,),
        in_specs=[
            pl.BlockSpec((EB, D), lambda i: (i, 0)),
            pl.BlockSpec((EB, 2 * D), lambda i: (i, 0)),
            pl.BlockSpec((EB, H), lambda i: (i, 0)),
            pl.BlockSpec((EB, 1), lambda i: (i, 0)),
            pl.BlockSpec((D, H), lambda i: (0, 0)),
        ],
        out_specs=pl.BlockSpec((EB, CW), lambda i: (i, 0)),
        out_shape=jax.ShapeDtypeStruct((E, CW), f32),
    )(qe, kespe, bias_t, dist_c, gmat)

    acc2 = _sc_scatter_add(vals, row_i, jnp.zeros((NPAD, CW), f32))
    acc3 = acc2.reshape(2, NPAD, CW)

    pos16 = jnp.concatenate([pos.astype(f32), jnp.zeros((N, 13), f32)], axis=1)
    w1 = W_in[:, :D]
    perm = jnp.array([D + 3 * h + c for c in range(3) for h in range(H)],
                     jnp.int32)
    w2 = W_in[:, perm]
    out = pl.pallas_call(
        _k3_body,
        grid=(N // RB,),
        in_specs=[
            pl.BlockSpec((1, RB, CW), lambda i: (0, i, 0)),
            pl.BlockSpec((1, RB, CW), lambda i: (1, i, 0)),
            pl.BlockSpec((RB, D), lambda i: (i, 0)),
            pl.BlockSpec((RB, D), lambda i: (i, 0)),
            pl.BlockSpec((RB, 16), lambda i: (i, 0)),
            pl.BlockSpec((HID, D), lambda i: (0, 0)),
            pl.BlockSpec((HID, 24), lambda i: (0, 0)),
            pl.BlockSpec((1, HID), lambda i: (0, 0)),
            pl.BlockSpec((D, HID), lambda i: (0, 0)),
            pl.BlockSpec((1, D), lambda i: (0, 0)),
        ],
        out_specs=pl.BlockSpec((RB, D), lambda i: (i, 0)),
        out_shape=jax.ShapeDtypeStruct((N, D), f32),
    )(acc3, acc3, x, zm, pos16, w1, w2, row1(b_in), W_out, row1(b_out))
    return out
